# baseline (speedup 1.0000x reference)
import functools
import jax
import jax.numpy as jnp
from jax import lax
from jax.experimental import pallas as pl
from jax.experimental.pallas import tpu as pltpu
from jax.experimental.pallas import tpu_sc as plsc

N = 100000
E = 1600000
G = 1024
D = 64
NC, NS = 2, 16
NW = NC * NS
NP = 102400
NAGG = 101376
RNG = 16896
RPAD = 16960
DUMMY = 16896
ASTRIPE = RPAD // NS
OSTRIPE = RNG // NS
EPT = E // NS
CE = 2000
NV = CE // 16
NCH = EPT // CE
FL = 128
NDCH = E // FL
DBASE = NDCH // NW
DEXTRA = NDCH - DBASE * NW
DSTRIPE = NP // NS
PRT = NP // NW
NPCH = PRT // FL
PSTRIPE = 1040 // NS
BM = 1024

_f32 = jnp.float32
_i32 = jnp.int32
_scp = pltpu.CompilerParams(use_tc_tiling_on_sc=False, needs_layout_passes=False)
_mesh = plsc.VectorSubcoreMesh(core_axis_name="c", subcore_axis_name="s", num_cores=NC, num_subcores=NS)


NDR = NP // 16
DST16 = NDR // NS


@functools.partial(
    pl.kernel, mesh=_mesh, compiler_params=_scp,
    out_type=jax.ShapeDtypeStruct((NC, NDR, 16), _f32),
    scratch_types=[
        pltpu.VMEM_SHARED((NDR, 16), _f32),
        pltpu.VMEM((FL,), _i32),
        pltpu.VMEM((FL, 16), _f32),
        pltpu.VMEM((FL,), _i32),
    ],
)
def _k_deg(dst2d, zoh_hbm, zcol_hbm, out, dacc, sidx, ohb, dbuf):
    c = lax.axis_index("c")
    s = lax.axis_index("s")
    w = s * NC + c
    iota16 = lax.iota(_i32, 16)
    ones16 = jnp.ones((16,), _f32)
    zeros16 = jnp.zeros((16,), _f32)
    pltpu.sync_copy(zcol_hbm, dacc.at[pl.ds(s * DST16, DST16)])
    pltpu.sync_copy(zoh_hbm, ohb)
    plsc.subcore_barrier()
    n = DBASE + jnp.where(w < DEXTRA, 1, 0)
    start = w * DBASE + jnp.minimum(w, DEXTRA)

    def body(k, carry):
        pltpu.sync_copy(dst2d.at[start + k], dbuf)
        for j in range(8):
            d16 = dbuf[pl.ds(j * 16, 16)]
            sidx[pl.ds(j * 16, 16)] = d16 >> 4
            plsc.store_scatter(ohb, [j * 16 + iota16, d16 & 15], ones16)
        pltpu.sync_copy(ohb, dacc.at[sidx], add=True)
        for j in range(8):
            d16 = dbuf[pl.ds(j * 16, 16)]
            plsc.store_scatter(ohb, [j * 16 + iota16, d16 & 15], zeros16)
        return carry

    lax.fori_loop(0, n, body, 0)
    plsc.subcore_barrier()
    pltpu.sync_copy(dacc.at[pl.ds(s * DST16, DST16)],
                    out.at[c, pl.ds(s * DST16, DST16)])


NQ = 4


@functools.partial(
    pl.kernel, mesh=_mesh, compiler_params=_scp,
    out_type=jax.ShapeDtypeStruct((NP, D), _f32),
    scratch_types=[
        pltpu.VMEM_SHARED((RPAD, D), _f32),
        pltpu.VMEM((CE,), _i32), pltpu.VMEM((CE,), _i32),
        pltpu.VMEM((CE,), _i32), pltpu.VMEM((CE,), _i32),
        pltpu.VMEM((CE + 16,), _i32),
        pltpu.VMEM((CE + 16,), _i32),
        pltpu.VMEM((NQ, FL), _i32),
        pltpu.VMEM((NQ, FL), _i32),
        pltpu.VMEM((NQ, FL, D), _f32),
        pltpu.SemaphoreType.DMA,
        pltpu.SemaphoreType.DMA,
        pltpu.SemaphoreType.DMA,
        pltpu.SemaphoreType.DMA,
    ],
)
def _k_agg(y_hbm, src_hbm, dst_hbm, zrow_hbm, out,
           acc, es0, ed0, es1, ed1, stg_s, stg_d,
           gidx, sidx, rows, semg, sems, see0, see1):
    c = lax.axis_index("c")
    s = lax.axis_index("s")
    iota16 = lax.iota(_i32, 16)
    esb = (es0, es1)
    edb = (ed0, ed1)
    seb = (see0, see1)

    def load_chunk(k, b):
        off = s * EPT + k * CE
        pltpu.make_async_copy(src_hbm.at[pl.ds(off, CE)], esb[b], seb[b]).start()
        pltpu.make_async_copy(dst_hbm.at[pl.ds(off, CE)], edb[b], seb[b]).start()

    def wait_chunk(b):
        pltpu.make_async_copy(src_hbm.at[pl.ds(0, CE)], esb[b], seb[b]).wait()
        pltpu.make_async_copy(dst_hbm.at[pl.ds(0, CE)], edb[b], seb[b]).wait()

    for p in range(3):
        r = 2 * p + c
        lo = r * RNG
        pltpu.sync_copy(zrow_hbm, acc.at[pl.ds(s * ASTRIPE, ASTRIPE)])
        plsc.subcore_barrier()
        load_chunk(0, 0)

        def chunk_pair(kp, carry, lo=lo):
            for b in range(2):
                k = kp * 2 + b

                @pl.when(k + 1 < NCH)
                def _(k=k, b=b):
                    load_chunk(k + 1, 1 - b)

                wait_chunk(b)
                esrc = esb[b]
                edst = edb[b]

                def scan_body(v, posv, esrc=esrc, edst=edst):
                    s16 = esrc[pl.ds(v * 16, 16)]
                    dl = edst[pl.ds(v * 16, 16)] - lo
                    m = (dl >= 0) & (dl < RNG)
                    cs = plsc.cumsum(jnp.where(m, 1, 0))
                    addr = posv + cs - 1
                    plsc.store_scatter(stg_s, [addr], s16, mask=m)
                    plsc.store_scatter(stg_d, [addr], dl, mask=m)
                    return posv + plsc.all_reduce_population_count(m)

                posv = lax.fori_loop(0, NV, scan_body, jnp.zeros((16,), _i32))
                pos = jnp.max(posv)
                nfl = (pos + FL - 1) // FL

                def superflush(fq, carry2, pos=pos, nfl=nfl):
                    base0 = fq * NQ * FL
                    for q in range(NQ):

                        @pl.when(base0 + q * FL < pos)
                        def _(q=q, base0=base0):
                            base = base0 + q * FL
                            for j in range(8):
                                lanes = base + j * 16 + iota16
                                mm = lanes < pos
                                sv = stg_s[pl.ds(base + j * 16, 16)]
                                dv = stg_d[pl.ds(base + j * 16, 16)]
                                gidx[q, pl.ds(j * 16, 16)] = jnp.where(mm, sv, 0)
                                sidx[q, pl.ds(j * 16, 16)] = jnp.where(mm, dv,
                                                                       DUMMY)
                            pltpu.make_async_copy(
                                y_hbm.at[gidx.at[q]], rows.at[q], semg).start()
                    for q in range(NQ):

                        @pl.when(base0 + q * FL < pos)
                        def _(q=q):
                            pltpu.make_async_copy(
                                y_hbm.at[gidx.at[q]], rows.at[q], semg).wait()
                            pltpu.make_async_copy(
                                rows.at[q], acc.at[sidx.at[q]],
                                sems).start(add=True)
                    for q in range(NQ):

                        @pl.when(base0 + q * FL < pos)
                        def _(q=q):
                            pltpu.make_async_copy(
                                rows.at[q], acc.at[sidx.at[q]], sems).wait()
                    return carry2

                lax.fori_loop(0, (nfl + NQ - 1) // NQ, superflush, 0)
            return carry

        lax.fori_loop(0, NCH // 2, chunk_pair, 0)
        plsc.subcore_barrier()
        pltpu.sync_copy(acc.at[pl.ds(s * OSTRIPE, OSTRIPE)],
                        out.at[pl.ds(lo + s * OSTRIPE, OSTRIPE)])
        plsc.subcore_barrier()


@functools.partial(
    pl.kernel, mesh=_mesh, compiler_params=_scp,
    out_type=jax.ShapeDtypeStruct((NC, G, 128), _f32),
    scratch_types=[
        pltpu.VMEM_SHARED((1040, 128), _f32),
        pltpu.VMEM((NPCH, FL), _i32),
        pltpu.VMEM((FL, 128), _f32),
    ],
)
def _k_pool(o3e, batch2d, zpool_hbm, out, pacc, pbuf, rows):
    c = lax.axis_index("c")
    s = lax.axis_index("s")
    w = s * NC + c
    pltpu.sync_copy(zpool_hbm, pacc.at[pl.ds(s * PSTRIPE, PSTRIPE)])
    plsc.subcore_barrier()
    pltpu.sync_copy(batch2d.at[pl.ds(w * NPCH, NPCH)], pbuf)

    def body(k, carry):
        off = w * PRT + k * FL
        pltpu.sync_copy(o3e.at[pl.ds(off, FL)], rows)
        pltpu.sync_copy(rows, pacc.at[pbuf.at[k]], add=True)
        return carry

    lax.fori_loop(0, NPCH, body, 0)
    plsc.subcore_barrier()
    pltpu.sync_copy(pacc.at[pl.ds(s * 64, 64)],
                    out.at[c, pl.ds(s * 64, 64)])


def _k_prep(degp, xp, w1p):
    def body(dg_ref, x_ref, w_ref, dis_ref, y_ref):
        cnt = dg_ref[0] + dg_ref[1]
        dis = lax.rsqrt(cnt + 1.0)
        y = jnp.dot(x_ref[...], w_ref[...], preferred_element_type=_f32)
        dis_ref[...] = dis
        y_ref[...] = y * dis

    return pl.pallas_call(
        body,
        grid=(NP // BM,),
        in_specs=[
            pl.BlockSpec((2, BM, 1), lambda i: (0, i, 0)),
            pl.BlockSpec((BM, D), lambda i: (i, 0)),
            pl.BlockSpec((D, D), lambda i: (0, 0)),
        ],
        out_specs=[
            pl.BlockSpec((BM, 1), lambda i: (i, 0)),
            pl.BlockSpec((BM, D), lambda i: (i, 0)),
        ],
        out_shape=[
            jax.ShapeDtypeStruct((NP, 1), _f32),
            jax.ShapeDtypeStruct((NP, D), _f32),
        ],
    )(degp, xp, w1p)


def _k_mid(agg, y, dis, wnext, b):
    def body(a_ref, y_ref, d_ref, w_ref, b_ref, o_ref):
        dis = d_ref[...]
        h = jax.nn.relu(dis * (a_ref[...] + y_ref[...]) + b_ref[...])
        o_ref[...] = dis * jnp.dot(h, w_ref[...],
                                   preferred_element_type=_f32)

    return pl.pallas_call(
        body,
        grid=(NP // BM,),
        in_specs=[
            pl.BlockSpec((BM, D), lambda i: (i, 0)),
            pl.BlockSpec((BM, D), lambda i: (i, 0)),
            pl.BlockSpec((BM, 1), lambda i: (i, 0)),
            pl.BlockSpec((D, D), lambda i: (0, 0)),
            pl.BlockSpec((1, D), lambda i: (0, 0)),
        ],
        out_specs=pl.BlockSpec((BM, D), lambda i: (i, 0)),
        out_shape=jax.ShapeDtypeStruct((NP, D), _f32),
    )(agg, y, dis, wnext, b)


def _k_l3(agg, y, dis, b):
    def body(a_ref, y_ref, d_ref, b_ref, o_ref):
        v = d_ref[...] * (a_ref[...] + y_ref[...]) + b_ref[...]
        o_ref[...] = jnp.concatenate(
            [v, jnp.ones((BM, 1), _f32), jnp.zeros((BM, 63), _f32)], axis=1)

    return pl.pallas_call(
        body,
        grid=(NP // BM,),
        in_specs=[
            pl.BlockSpec((BM, D), lambda i: (i, 0)),
            pl.BlockSpec((BM, D), lambda i: (i, 0)),
            pl.BlockSpec((BM, 1), lambda i: (i, 0)),
            pl.BlockSpec((1, D), lambda i: (0, 0)),
        ],
        out_specs=pl.BlockSpec((BM, 128), lambda i: (i, 0)),
        out_shape=jax.ShapeDtypeStruct((NP, 128), _f32),
    )(agg, y, dis, b)


def _k_fin(parts, wlp, blp):
    def body(p_ref, w_ref, b_ref, o_ref):
        sums = p_ref[0] + p_ref[1]
        cnt = jnp.maximum(sums[:, 64:65], 1.0)
        pooled = sums[:, :D] / cnt
        o_ref[...] = jnp.dot(pooled, w_ref[...],
                             preferred_element_type=_f32) + b_ref[...]

    return pl.pallas_call(
        body,
        in_specs=[
            pl.BlockSpec((NC, G, 128), lambda: (0, 0, 0)),
            pl.BlockSpec((D, 128), lambda: (0, 0)),
            pl.BlockSpec((1, 128), lambda: (0, 0)),
        ],
        out_specs=pl.BlockSpec((G, 128), lambda: (0, 0)),
        out_shape=jax.ShapeDtypeStruct((G, 128), _f32),
    )(parts, wlp, blp)


def kernel(x, edge_index, batch, W1, b1, W2, b2, W3, b3, Wl, bl):
    xp = jnp.zeros((NP, D), _f32).at[:N, :50].set(x)
    w1p = jnp.zeros((D, D), _f32).at[:50, :].set(W1)
    wlp = jnp.zeros((D, 128), _f32).at[:, :2].set(Wl)
    blp = jnp.zeros((1, 128), _f32).at[0, :2].set(bl)
    b1r = b1.reshape(1, D)
    b2r = b2.reshape(1, D)
    b3r = b3.reshape(1, D)
    src = edge_index[0]
    dst = edge_index[1]
    dst2d = dst.reshape(E // FL, FL)
    batch2d = jnp.concatenate(
        [batch, jnp.full((NP - N,), G, _i32)]).reshape(NP // FL, FL)
    zoh = jnp.zeros((FL, 16), _f32)
    zcol = jnp.zeros((DST16, 16), _f32)
    zrow = jnp.zeros((ASTRIPE, D), _f32)
    zpool = jnp.zeros((PSTRIPE, 128), _f32)

    degp = _k_deg(dst2d, zoh, zcol).reshape(NC, NP, 1)
    dis, y1 = _k_prep(degp, xp, w1p)
    a1 = _k_agg(y1, src, dst, zrow)
    y2 = _k_mid(a1, y1, dis, W2, b1r)
    a2 = _k_agg(y2, src, dst, zrow)
    y3 = _k_mid(a2, y2, dis, W3, b2r)
    a3 = _k_agg(y3, src, dst, zrow)
    o3e = _k_l3(a3, y3, dis, b3r)
    parts = _k_pool(o3e, batch2d, zpool)
    outp = _k_fin(parts, wlp, blp)
    return outp[:, :2]


# speedup vs baseline: 1.0016x; 1.0016x over previous
import functools
import jax
import jax.numpy as jnp
from jax import lax
from jax.experimental import pallas as pl
from jax.experimental.pallas import tpu as pltpu
from jax.experimental.pallas import tpu_sc as plsc

N = 100000
E = 1600000
G = 1024
D = 64
NC, NS = 2, 16
NW = NC * NS
NP = 102400
NAGG = 101376
RNG = 16896
RPAD = 16960
DUMMY = 16896
ASTRIPE = RPAD // NS
OSTRIPE = RNG // NS
EPT = E // NS
CE = 2000
NV = CE // 16
NCH = EPT // CE
FL = 128
NDCH = E // FL
DBASE = NDCH // NW
DEXTRA = NDCH - DBASE * NW
DSTRIPE = NP // NS
PRT = NP // NW
NPCH = PRT // FL
PSTRIPE = 1040 // NS
BM = 1024

_f32 = jnp.float32
_i32 = jnp.int32
_scp = pltpu.CompilerParams(use_tc_tiling_on_sc=False, needs_layout_passes=False)
_mesh = plsc.VectorSubcoreMesh(core_axis_name="c", subcore_axis_name="s", num_cores=NC, num_subcores=NS)


NDR = NP // 16
DST16 = NDR // NS


@functools.partial(
    pl.kernel, mesh=_mesh, compiler_params=_scp,
    out_type=jax.ShapeDtypeStruct((NC, NDR, 16), _f32),
    scratch_types=[
        pltpu.VMEM_SHARED((NDR, 16), _f32),
        pltpu.VMEM((FL,), _i32),
        pltpu.VMEM((FL, 16), _f32),
        pltpu.VMEM((FL,), _i32),
    ],
)
def _k_deg(dst2d, zoh_hbm, zcol_hbm, out, dacc, sidx, ohb, dbuf):
    c = lax.axis_index("c")
    s = lax.axis_index("s")
    w = s * NC + c
    iota16 = lax.iota(_i32, 16)
    ones16 = jnp.ones((16,), _f32)
    zeros16 = jnp.zeros((16,), _f32)
    pltpu.sync_copy(zcol_hbm, dacc.at[pl.ds(s * DST16, DST16)])
    pltpu.sync_copy(zoh_hbm, ohb)
    plsc.subcore_barrier()
    n = DBASE + jnp.where(w < DEXTRA, 1, 0)
    start = w * DBASE + jnp.minimum(w, DEXTRA)

    def body(k, carry):
        pltpu.sync_copy(dst2d.at[start + k], dbuf)
        for j in range(8):
            d16 = dbuf[pl.ds(j * 16, 16)]
            sidx[pl.ds(j * 16, 16)] = d16 >> 4
            plsc.store_scatter(ohb, [j * 16 + iota16, d16 & 15], ones16)
        pltpu.sync_copy(ohb, dacc.at[sidx], add=True)
        for j in range(8):
            d16 = dbuf[pl.ds(j * 16, 16)]
            plsc.store_scatter(ohb, [j * 16 + iota16, d16 & 15], zeros16)
        return carry

    lax.fori_loop(0, n, body, 0)
    plsc.subcore_barrier()
    pltpu.sync_copy(dacc.at[pl.ds(s * DST16, DST16)],
                    out.at[c, pl.ds(s * DST16, DST16)])


NQ = 4


@functools.partial(
    pl.kernel, mesh=_mesh, compiler_params=_scp,
    out_type=jax.ShapeDtypeStruct((NP, D), _f32),
    scratch_types=[
        pltpu.VMEM_SHARED((RPAD, D), _f32),
        pltpu.VMEM((CE,), _i32), pltpu.VMEM((CE,), _i32),
        pltpu.VMEM((CE,), _i32), pltpu.VMEM((CE,), _i32),
        pltpu.VMEM((CE + 16,), _i32),
        pltpu.VMEM((CE + 16,), _i32),
        pltpu.VMEM((NQ, FL), _i32),
        pltpu.VMEM((NQ, FL), _i32),
        pltpu.VMEM((NQ, FL, D), _f32),
        pltpu.SemaphoreType.DMA,
        pltpu.SemaphoreType.DMA,
        pltpu.SemaphoreType.DMA,
        pltpu.SemaphoreType.DMA,
    ],
)
def _k_agg(y_hbm, src_hbm, dst_hbm, zrow_hbm, out,
           acc, es0, ed0, es1, ed1, stg_s, stg_d,
           gidx, sidx, rows, semg, sems, see0, see1):
    c = lax.axis_index("c")
    s = lax.axis_index("s")
    iota16 = lax.iota(_i32, 16)
    esb = (es0, es1)
    edb = (ed0, ed1)
    seb = (see0, see1)

    def load_chunk(k, b):
        off = s * EPT + k * CE
        pltpu.make_async_copy(src_hbm.at[pl.ds(off, CE)], esb[b], seb[b]).start()
        pltpu.make_async_copy(dst_hbm.at[pl.ds(off, CE)], edb[b], seb[b]).start()

    def wait_chunk(b):
        pltpu.make_async_copy(src_hbm.at[pl.ds(0, CE)], esb[b], seb[b]).wait()
        pltpu.make_async_copy(dst_hbm.at[pl.ds(0, CE)], edb[b], seb[b]).wait()

    for p in range(3):
        r = 2 * p + c
        lo = r * RNG
        pltpu.sync_copy(zrow_hbm, acc.at[pl.ds(s * ASTRIPE, ASTRIPE)])
        plsc.subcore_barrier()
        load_chunk(0, 0)

        def chunk_pair(kp, carry, lo=lo):
            for b in range(2):
                k = kp * 2 + b

                @pl.when(k + 1 < NCH)
                def _(k=k, b=b):
                    load_chunk(k + 1, 1 - b)

                wait_chunk(b)
                esrc = esb[b]
                edst = edb[b]

                def scan_body(v, posv, esrc=esrc, edst=edst):
                    s16 = esrc[pl.ds(v * 16, 16)]
                    dl = edst[pl.ds(v * 16, 16)] - lo
                    m = (dl >= 0) & (dl < RNG)
                    cs = plsc.cumsum(jnp.where(m, 1, 0))
                    addr = posv + cs - 1
                    plsc.store_scatter(stg_s, [addr], s16, mask=m)
                    plsc.store_scatter(stg_d, [addr], dl, mask=m)
                    return posv + plsc.all_reduce_population_count(m)

                posv = plsc.parallel_loop(
                    0, NV, carry=jnp.zeros((16,), _i32), unroll=8)(scan_body)
                pos = jnp.max(posv)
                nfl = (pos + FL - 1) // FL

                def superflush(fq, carry2, pos=pos, nfl=nfl):
                    base0 = fq * NQ * FL
                    for q in range(NQ):

                        @pl.when(base0 + q * FL < pos)
                        def _(q=q, base0=base0):
                            base = base0 + q * FL
                            for j in range(8):
                                lanes = base + j * 16 + iota16
                                mm = lanes < pos
                                sv = stg_s[pl.ds(base + j * 16, 16)]
                                dv = stg_d[pl.ds(base + j * 16, 16)]
                                gidx[q, pl.ds(j * 16, 16)] = jnp.where(mm, sv, 0)
                                sidx[q, pl.ds(j * 16, 16)] = jnp.where(mm, dv,
                                                                       DUMMY)
                            pltpu.make_async_copy(
                                y_hbm.at[gidx.at[q]], rows.at[q], semg).start()
                    for q in range(NQ):

                        @pl.when(base0 + q * FL < pos)
                        def _(q=q):
                            pltpu.make_async_copy(
                                y_hbm.at[gidx.at[q]], rows.at[q], semg).wait()
                            pltpu.make_async_copy(
                                rows.at[q], acc.at[sidx.at[q]],
                                sems).start(add=True)
                    for q in range(NQ):

                        @pl.when(base0 + q * FL < pos)
                        def _(q=q):
                            pltpu.make_async_copy(
                                rows.at[q], acc.at[sidx.at[q]], sems).wait()
                    return carry2

                lax.fori_loop(0, (nfl + NQ - 1) // NQ, superflush, 0)
            return carry

        lax.fori_loop(0, NCH // 2, chunk_pair, 0)
        plsc.subcore_barrier()
        pltpu.sync_copy(acc.at[pl.ds(s * OSTRIPE, OSTRIPE)],
                        out.at[pl.ds(lo + s * OSTRIPE, OSTRIPE)])
        plsc.subcore_barrier()


@functools.partial(
    pl.kernel, mesh=_mesh, compiler_params=_scp,
    out_type=jax.ShapeDtypeStruct((NC, G, 128), _f32),
    scratch_types=[
        pltpu.VMEM_SHARED((1040, 128), _f32),
        pltpu.VMEM((NPCH, FL), _i32),
        pltpu.VMEM((FL, 128), _f32),
    ],
)
def _k_pool(o3e, batch2d, zpool_hbm, out, pacc, pbuf, rows):
    c = lax.axis_index("c")
    s = lax.axis_index("s")
    w = s * NC + c
    pltpu.sync_copy(zpool_hbm, pacc.at[pl.ds(s * PSTRIPE, PSTRIPE)])
    plsc.subcore_barrier()
    pltpu.sync_copy(batch2d.at[pl.ds(w * NPCH, NPCH)], pbuf)

    def body(k, carry):
        off = w * PRT + k * FL
        pltpu.sync_copy(o3e.at[pl.ds(off, FL)], rows)
        pltpu.sync_copy(rows, pacc.at[pbuf.at[k]], add=True)
        return carry

    lax.fori_loop(0, NPCH, body, 0)
    plsc.subcore_barrier()
    pltpu.sync_copy(pacc.at[pl.ds(s * 64, 64)],
                    out.at[c, pl.ds(s * 64, 64)])


def _k_prep(degp, xp, w1p):
    def body(dg_ref, x_ref, w_ref, dis_ref, y_ref):
        cnt = dg_ref[0] + dg_ref[1]
        dis = lax.rsqrt(cnt + 1.0)
        y = jnp.dot(x_ref[...], w_ref[...], preferred_element_type=_f32)
        dis_ref[...] = dis
        y_ref[...] = y * dis

    return pl.pallas_call(
        body,
        grid=(NP // BM,),
        in_specs=[
            pl.BlockSpec((2, BM, 1), lambda i: (0, i, 0)),
            pl.BlockSpec((BM, D), lambda i: (i, 0)),
            pl.BlockSpec((D, D), lambda i: (0, 0)),
        ],
        out_specs=[
            pl.BlockSpec((BM, 1), lambda i: (i, 0)),
            pl.BlockSpec((BM, D), lambda i: (i, 0)),
        ],
        out_shape=[
            jax.ShapeDtypeStruct((NP, 1), _f32),
            jax.ShapeDtypeStruct((NP, D), _f32),
        ],
    )(degp, xp, w1p)


def _k_mid(agg, y, dis, wnext, b):
    def body(a_ref, y_ref, d_ref, w_ref, b_ref, o_ref):
        dis = d_ref[...]
        h = jax.nn.relu(dis * (a_ref[...] + y_ref[...]) + b_ref[...])
        o_ref[...] = dis * jnp.dot(h, w_ref[...],
                                   preferred_element_type=_f32)

    return pl.pallas_call(
        body,
        grid=(NP // BM,),
        in_specs=[
            pl.BlockSpec((BM, D), lambda i: (i, 0)),
            pl.BlockSpec((BM, D), lambda i: (i, 0)),
            pl.BlockSpec((BM, 1), lambda i: (i, 0)),
            pl.BlockSpec((D, D), lambda i: (0, 0)),
            pl.BlockSpec((1, D), lambda i: (0, 0)),
        ],
        out_specs=pl.BlockSpec((BM, D), lambda i: (i, 0)),
        out_shape=jax.ShapeDtypeStruct((NP, D), _f32),
    )(agg, y, dis, wnext, b)


def _k_l3(agg, y, dis, b):
    def body(a_ref, y_ref, d_ref, b_ref, o_ref):
        v = d_ref[...] * (a_ref[...] + y_ref[...]) + b_ref[...]
        o_ref[...] = jnp.concatenate(
            [v, jnp.ones((BM, 1), _f32), jnp.zeros((BM, 63), _f32)], axis=1)

    return pl.pallas_call(
        body,
        grid=(NP // BM,),
        in_specs=[
            pl.BlockSpec((BM, D), lambda i: (i, 0)),
            pl.BlockSpec((BM, D), lambda i: (i, 0)),
            pl.BlockSpec((BM, 1), lambda i: (i, 0)),
            pl.BlockSpec((1, D), lambda i: (0, 0)),
        ],
        out_specs=pl.BlockSpec((BM, 128), lambda i: (i, 0)),
        out_shape=jax.ShapeDtypeStruct((NP, 128), _f32),
    )(agg, y, dis, b)


def _k_fin(parts, wlp, blp):
    def body(p_ref, w_ref, b_ref, o_ref):
        sums = p_ref[0] + p_ref[1]
        cnt = jnp.maximum(sums[:, 64:65], 1.0)
        pooled = sums[:, :D] / cnt
        o_ref[...] = jnp.dot(pooled, w_ref[...],
                             preferred_element_type=_f32) + b_ref[...]

    return pl.pallas_call(
        body,
        in_specs=[
            pl.BlockSpec((NC, G, 128), lambda: (0, 0, 0)),
            pl.BlockSpec((D, 128), lambda: (0, 0)),
            pl.BlockSpec((1, 128), lambda: (0, 0)),
        ],
        out_specs=pl.BlockSpec((G, 128), lambda: (0, 0)),
        out_shape=jax.ShapeDtypeStruct((G, 128), _f32),
    )(parts, wlp, blp)


def kernel(x, edge_index, batch, W1, b1, W2, b2, W3, b3, Wl, bl):
    xp = jnp.zeros((NP, D), _f32).at[:N, :50].set(x)
    w1p = jnp.zeros((D, D), _f32).at[:50, :].set(W1)
    wlp = jnp.zeros((D, 128), _f32).at[:, :2].set(Wl)
    blp = jnp.zeros((1, 128), _f32).at[0, :2].set(bl)
    b1r = b1.reshape(1, D)
    b2r = b2.reshape(1, D)
    b3r = b3.reshape(1, D)
    src = edge_index[0]
    dst = edge_index[1]
    dst2d = dst.reshape(E // FL, FL)
    batch2d = jnp.concatenate(
        [batch, jnp.full((NP - N,), G, _i32)]).reshape(NP // FL, FL)
    zoh = jnp.zeros((FL, 16), _f32)
    zcol = jnp.zeros((DST16, 16), _f32)
    zrow = jnp.zeros((ASTRIPE, D), _f32)
    zpool = jnp.zeros((PSTRIPE, 128), _f32)

    degp = _k_deg(dst2d, zoh, zcol).reshape(NC, NP, 1)
    dis, y1 = _k_prep(degp, xp, w1p)
    a1 = _k_agg(y1, src, dst, zrow)
    y2 = _k_mid(a1, y1, dis, W2, b1r)
    a2 = _k_agg(y2, src, dst, zrow)
    y3 = _k_mid(a2, y2, dis, W3, b2r)
    a3 = _k_agg(y3, src, dst, zrow)
    o3e = _k_l3(a3, y3, dis, b3r)
    parts = _k_pool(o3e, batch2d, zpool)
    outp = _k_fin(parts, wlp, blp)
    return outp[:, :2]


# speedup vs baseline: 1.2543x; 1.2523x over previous
import functools
import jax
import jax.numpy as jnp
from jax import lax
from jax.experimental import pallas as pl
from jax.experimental.pallas import tpu as pltpu
from jax.experimental.pallas import tpu_sc as plsc

N = 100000
E = 1600000
G = 1024
D = 64
NC, NS = 2, 16
NW = NC * NS
NP = 102400
RNG = 25600
RPAD = 25616
DUMMY = 25600
ASTRIPE = RPAD // NS
OSTRIPE = RNG // NS
EPT = E // NS
CE = 2000
NV = CE // 16
NCH = EPT // CE
FL = 128
NDCH = E // FL
DBASE = NDCH // NW
DEXTRA = NDCH - DBASE * NW
DSTRIPE = NP // NS
PRT = NP // NW
NPCH = PRT // FL
PSTRIPE = 1040 // NS
BM = 1024

_f32 = jnp.float32
_i32 = jnp.int32
_scp = pltpu.CompilerParams(use_tc_tiling_on_sc=False, needs_layout_passes=False)
_mesh = plsc.VectorSubcoreMesh(core_axis_name="c", subcore_axis_name="s", num_cores=NC, num_subcores=NS)


NDR = NP // 16
DST16 = NDR // NS


@functools.partial(
    pl.kernel, mesh=_mesh, compiler_params=_scp,
    out_type=jax.ShapeDtypeStruct((NC, NDR, 16), _f32),
    scratch_types=[
        pltpu.VMEM_SHARED((NDR, 16), _f32),
        pltpu.VMEM((FL,), _i32),
        pltpu.VMEM((FL, 16), _f32),
        pltpu.VMEM((FL,), _i32),
    ],
)
def _k_deg(dst2d, zoh_hbm, zcol_hbm, out, dacc, sidx, ohb, dbuf):
    c = lax.axis_index("c")
    s = lax.axis_index("s")
    w = s * NC + c
    iota16 = lax.iota(_i32, 16)
    ones16 = jnp.ones((16,), _f32)
    zeros16 = jnp.zeros((16,), _f32)
    pltpu.sync_copy(zcol_hbm, dacc.at[pl.ds(s * DST16, DST16)])
    pltpu.sync_copy(zoh_hbm, ohb)
    plsc.subcore_barrier()
    n = DBASE + jnp.where(w < DEXTRA, 1, 0)
    start = w * DBASE + jnp.minimum(w, DEXTRA)

    def body(k, carry):
        pltpu.sync_copy(dst2d.at[start + k], dbuf)
        for j in range(8):
            d16 = dbuf[pl.ds(j * 16, 16)]
            sidx[pl.ds(j * 16, 16)] = d16 >> 4
            plsc.store_scatter(ohb, [j * 16 + iota16, d16 & 15], ones16)
        pltpu.sync_copy(ohb, dacc.at[sidx], add=True)
        for j in range(8):
            d16 = dbuf[pl.ds(j * 16, 16)]
            plsc.store_scatter(ohb, [j * 16 + iota16, d16 & 15], zeros16)
        return carry

    lax.fori_loop(0, n, body, 0)
    plsc.subcore_barrier()
    pltpu.sync_copy(dacc.at[pl.ds(s * DST16, DST16)],
                    out.at[c, pl.ds(s * DST16, DST16)])


NQ = 2
SCAP = 2048
SNR = SCAP // FL


@functools.partial(
    pl.kernel, mesh=_mesh, compiler_params=_scp,
    out_type=jax.ShapeDtypeStruct((NP, D), _f32),
    scratch_types=[
        pltpu.VMEM_SHARED((RPAD, D), _f32),
        pltpu.VMEM((CE,), _i32), pltpu.VMEM((CE,), _i32),
        pltpu.VMEM((CE,), _i32), pltpu.VMEM((CE,), _i32),
        pltpu.VMEM((SCAP,), _i32),
        pltpu.VMEM((SNR, FL), _i32),
        pltpu.VMEM((NQ, FL, D), _f32),
        pltpu.SemaphoreType.DMA,
        pltpu.SemaphoreType.DMA,
        pltpu.SemaphoreType.DMA,
        pltpu.SemaphoreType.DMA,
    ],
)
def _k_agg(y_hbm, src_hbm, dst_hbm, zrow_hbm, out,
           acc, es0, ed0, es1, ed1, stg_s, stg_d, rows,
           semg, sems, see0, see1):
    c = lax.axis_index("c")
    s = lax.axis_index("s")
    iota16 = lax.iota(_i32, 16)
    esb = (es0, es1)
    edb = (ed0, ed1)
    seb = (see0, see1)

    def load_chunk(k, b):
        off = s * EPT + k * CE
        pltpu.make_async_copy(src_hbm.at[pl.ds(off, CE)], esb[b], seb[b]).start()
        pltpu.make_async_copy(dst_hbm.at[pl.ds(off, CE)], edb[b], seb[b]).start()

    def wait_chunk(b):
        pltpu.make_async_copy(src_hbm.at[pl.ds(0, CE)], esb[b], seb[b]).wait()
        pltpu.make_async_copy(dst_hbm.at[pl.ds(0, CE)], edb[b], seb[b]).wait()

    for p in range(2):
        r = 2 * p + c
        lo = r * RNG
        pltpu.sync_copy(zrow_hbm, acc.at[pl.ds(s * ASTRIPE, ASTRIPE)])
        plsc.subcore_barrier()
        load_chunk(0, 0)

        def chunk_pair(kp, carry, lo=lo):
            for b in range(2):
                k = kp * 2 + b

                @pl.when(k + 1 < NCH)
                def _(k=k, b=b):
                    load_chunk(k + 1, 1 - b)

                wait_chunk(b)
                esrc = esb[b]
                edst = edb[b]

                def scan_body(v, posv, esrc=esrc, edst=edst):
                    s16 = esrc[pl.ds(v * 16, 16)]
                    dl = edst[pl.ds(v * 16, 16)] - lo
                    m = (dl >= 0) & (dl < RNG)
                    cs = plsc.cumsum(jnp.where(m, 1, 0))
                    addr = posv + cs - 1
                    plsc.store_scatter(stg_s, [addr], s16, mask=m)
                    plsc.store_scatter(stg_d, [addr >> 7, addr & 127], dl,
                                       mask=m)
                    return posv + plsc.all_reduce_population_count(m)

                posv = plsc.parallel_loop(
                    0, NV, carry=jnp.zeros((16,), _i32), unroll=8)(scan_body)
                pos = jnp.max(posv)
                nfl = (pos + FL - 1) // FL
                zfill = jnp.zeros((16,), _i32)
                dfill = jnp.full((16,), DUMMY, _i32)
                for j in range(9):
                    a = pos + j * 16 + iota16
                    mf = a < SCAP
                    plsc.store_scatter(stg_s, [a], zfill, mask=mf)
                    plsc.store_scatter(stg_d, [a >> 7, a & 127], dfill,
                                       mask=mf)

                def superflush(fq, carry2, nfl=nfl):
                    for q in range(NQ):
                        f = fq * NQ + q

                        @pl.when(f < nfl)
                        def _(f=f, q=q):
                            @pl.when(f >= NQ)
                            def _():
                                pltpu.make_async_copy(
                                    rows.at[q], acc.at[stg_d.at[0]],
                                    sems).wait()

                            pltpu.make_async_copy(
                                y_hbm.at[stg_s.at[pl.ds(f * FL, FL)]],
                                rows.at[q], semg).start()
                    for q in range(NQ):
                        f = fq * NQ + q

                        @pl.when(f < nfl)
                        def _(f=f, q=q):
                            pltpu.make_async_copy(
                                y_hbm.at[stg_s.at[pl.ds(f * FL, FL)]],
                                rows.at[q], semg).wait()
                            pltpu.make_async_copy(
                                rows.at[q], acc.at[stg_d.at[f]],
                                sems).start(add=True)
                    return carry2

                lax.fori_loop(0, (nfl + NQ - 1) // NQ, superflush, 0)
                for q in range(NQ):

                    @pl.when(nfl > q)
                    def _(q=q):
                        pltpu.make_async_copy(
                            rows.at[q], acc.at[stg_d.at[0]], sems).wait()
            return carry

        lax.fori_loop(0, NCH // 2, chunk_pair, 0)
        plsc.subcore_barrier()
        pltpu.sync_copy(acc.at[pl.ds(s * OSTRIPE, OSTRIPE)],
                        out.at[pl.ds(lo + s * OSTRIPE, OSTRIPE)])
        plsc.subcore_barrier()


@functools.partial(
    pl.kernel, mesh=_mesh, compiler_params=_scp,
    out_type=jax.ShapeDtypeStruct((NC, G, 128), _f32),
    scratch_types=[
        pltpu.VMEM_SHARED((1040, 128), _f32),
        pltpu.VMEM((NPCH, FL), _i32),
        pltpu.VMEM((FL, 128), _f32),
    ],
)
def _k_pool(o3e, batch2d, zpool_hbm, out, pacc, pbuf, rows):
    c = lax.axis_index("c")
    s = lax.axis_index("s")
    w = s * NC + c
    pltpu.sync_copy(zpool_hbm, pacc.at[pl.ds(s * PSTRIPE, PSTRIPE)])
    plsc.subcore_barrier()
    pltpu.sync_copy(batch2d.at[pl.ds(w * NPCH, NPCH)], pbuf)

    def body(k, carry):
        off = w * PRT + k * FL
        pltpu.sync_copy(o3e.at[pl.ds(off, FL)], rows)
        pltpu.sync_copy(rows, pacc.at[pbuf.at[k]], add=True)
        return carry

    lax.fori_loop(0, NPCH, body, 0)
    plsc.subcore_barrier()
    pltpu.sync_copy(pacc.at[pl.ds(s * 64, 64)],
                    out.at[c, pl.ds(s * 64, 64)])


def _k_prep(degp, xp, w1p):
    def body(dg_ref, x_ref, w_ref, dis_ref, y_ref):
        cnt = dg_ref[0] + dg_ref[1]
        dis = lax.rsqrt(cnt + 1.0)
        y = jnp.dot(x_ref[...], w_ref[...], preferred_element_type=_f32)
        dis_ref[...] = dis
        y_ref[...] = y * dis

    return pl.pallas_call(
        body,
        grid=(NP // BM,),
        in_specs=[
            pl.BlockSpec((2, BM, 1), lambda i: (0, i, 0)),
            pl.BlockSpec((BM, D), lambda i: (i, 0)),
            pl.BlockSpec((D, D), lambda i: (0, 0)),
        ],
        out_specs=[
            pl.BlockSpec((BM, 1), lambda i: (i, 0)),
            pl.BlockSpec((BM, D), lambda i: (i, 0)),
        ],
        out_shape=[
            jax.ShapeDtypeStruct((NP, 1), _f32),
            jax.ShapeDtypeStruct((NP, D), _f32),
        ],
    )(degp, xp, w1p)


def _k_mid(agg, y, dis, wnext, b):
    def body(a_ref, y_ref, d_ref, w_ref, b_ref, o_ref):
        dis = d_ref[...]
        h = jax.nn.relu(dis * (a_ref[...] + y_ref[...]) + b_ref[...])
        o_ref[...] = dis * jnp.dot(h, w_ref[...],
                                   preferred_element_type=_f32)

    return pl.pallas_call(
        body,
        grid=(NP // BM,),
        in_specs=[
            pl.BlockSpec((BM, D), lambda i: (i, 0)),
            pl.BlockSpec((BM, D), lambda i: (i, 0)),
            pl.BlockSpec((BM, 1), lambda i: (i, 0)),
            pl.BlockSpec((D, D), lambda i: (0, 0)),
            pl.BlockSpec((1, D), lambda i: (0, 0)),
        ],
        out_specs=pl.BlockSpec((BM, D), lambda i: (i, 0)),
        out_shape=jax.ShapeDtypeStruct((NP, D), _f32),
    )(agg, y, dis, wnext, b)


def _k_l3(agg, y, dis, b):
    def body(a_ref, y_ref, d_ref, b_ref, o_ref):
        v = d_ref[...] * (a_ref[...] + y_ref[...]) + b_ref[...]
        o_ref[...] = jnp.concatenate(
            [v, jnp.ones((BM, 1), _f32), jnp.zeros((BM, 63), _f32)], axis=1)

    return pl.pallas_call(
        body,
        grid=(NP // BM,),
        in_specs=[
            pl.BlockSpec((BM, D), lambda i: (i, 0)),
            pl.BlockSpec((BM, D), lambda i: (i, 0)),
            pl.BlockSpec((BM, 1), lambda i: (i, 0)),
            pl.BlockSpec((1, D), lambda i: (0, 0)),
        ],
        out_specs=pl.BlockSpec((BM, 128), lambda i: (i, 0)),
        out_shape=jax.ShapeDtypeStruct((NP, 128), _f32),
    )(agg, y, dis, b)


def _k_fin(parts, wlp, blp):
    def body(p_ref, w_ref, b_ref, o_ref):
        sums = p_ref[0] + p_ref[1]
        cnt = jnp.maximum(sums[:, 64:65], 1.0)
        pooled = sums[:, :D] / cnt
        o_ref[...] = jnp.dot(pooled, w_ref[...],
                             preferred_element_type=_f32) + b_ref[...]

    return pl.pallas_call(
        body,
        in_specs=[
            pl.BlockSpec((NC, G, 128), lambda: (0, 0, 0)),
            pl.BlockSpec((D, 128), lambda: (0, 0)),
            pl.BlockSpec((1, 128), lambda: (0, 0)),
        ],
        out_specs=pl.BlockSpec((G, 128), lambda: (0, 0)),
        out_shape=jax.ShapeDtypeStruct((G, 128), _f32),
    )(parts, wlp, blp)


def kernel(x, edge_index, batch, W1, b1, W2, b2, W3, b3, Wl, bl):
    xp = jnp.zeros((NP, D), _f32).at[:N, :50].set(x)
    w1p = jnp.zeros((D, D), _f32).at[:50, :].set(W1)
    wlp = jnp.zeros((D, 128), _f32).at[:, :2].set(Wl)
    blp = jnp.zeros((1, 128), _f32).at[0, :2].set(bl)
    b1r = b1.reshape(1, D)
    b2r = b2.reshape(1, D)
    b3r = b3.reshape(1, D)
    src = edge_index[0]
    dst = edge_index[1]
    dst2d = dst.reshape(E // FL, FL)
    batch2d = jnp.concatenate(
        [batch, jnp.full((NP - N,), G, _i32)]).reshape(NP // FL, FL)
    zoh = jnp.zeros((FL, 16), _f32)
    zcol = jnp.zeros((DST16, 16), _f32)
    zrow = jnp.zeros((ASTRIPE, D), _f32)
    zpool = jnp.zeros((PSTRIPE, 128), _f32)

    degp = _k_deg(dst2d, zoh, zcol).reshape(NC, NP, 1)
    dis, y1 = _k_prep(degp, xp, w1p)
    a1 = _k_agg(y1, src, dst, zrow)
    y2 = _k_mid(a1, y1, dis, W2, b1r)
    a2 = _k_agg(y2, src, dst, zrow)
    y3 = _k_mid(a2, y2, dis, W3, b2r)
    a3 = _k_agg(y3, src, dst, zrow)
    o3e = _k_l3(a3, y3, dis, b3r)
    parts = _k_pool(o3e, batch2d, zpool)
    outp = _k_fin(parts, wlp, blp)
    return outp[:, :2]
